# baseline (speedup 1.0000x reference)
import jax
import jax.numpy as jnp
from jax.experimental import pallas as pl
from jax.experimental.pallas import tpu as pltpu

_LOW = 7
_TILE = 16


def _cmpex_reshape(w, jp):
    n, c = w.shape
    g = n // (2 * jp)
    w4 = w.reshape(g, 2, jp, c)
    a, b = w4[:, 0], w4[:, 1]
    lo, hi = jnp.minimum(a, b), jnp.maximum(a, b)
    return jnp.concatenate([lo[:, None], hi[:, None]], axis=1).reshape(n, c)


def _cmpex_roll(w, jp, row):
    n, c = w.shape
    w3 = w.reshape(n // _TILE, _TILE, c)
    down = pltpu.roll(w3, _TILE - jp, 1)
    lo = jnp.minimum(w3, down)
    hi = jnp.maximum(w3, down)
    hi_up = pltpu.roll(hi, jp, 1)
    t = jax.lax.broadcasted_iota(jnp.int32, (1, _TILE, 1), 1)
    is_a = (t & jp) == 0
    return jnp.where(is_a, lo, hi_up).reshape(n, c)


def _bitonic_sort_rot(v):
    n = v.shape[0]
    row = jax.lax.broadcasted_iota(jnp.int32, (n, 1), 0)
    i_row = ((row & (_TILE - 1)) << _LOW) | (row >> 4)

    def sign(k):
        if k >= n:
            return None
        return jnp.where((i_row & k) == 0, 1.0, -1.0).astype(v.dtype)

    def cmpex(w, j):
        b = j.bit_length() - 1
        if b < _LOW:
            return _cmpex_reshape(w, _TILE << b)
        return _cmpex_roll(w, 1 << (b - _LOW), row)

    w = v * sign(2)
    k = 2
    while k <= n:
        j = k // 2
        while j >= 1:
            w = cmpex(w, j)
            j //= 2
        s_old, s_new = sign(k), sign(2 * k)
        if s_new is not None:
            w = w * (s_old * s_new)
        elif k < n:
            w = w * s_old
        k *= 2
    return w


def _swd_body(x_ref, y_ref, tht_ref, s_ref):
    nb = x_ref.shape[0]
    p = tht_ref.shape[2]
    projs = []
    for i in range(nb):
        projs.append(jnp.dot(x_ref[i], tht_ref[i],
                             preferred_element_type=jnp.float32))
        projs.append(jnp.dot(y_ref[i], tht_ref[i],
                             preferred_element_type=jnp.float32))
    both = jnp.concatenate(projs, axis=1)
    srt = _bitonic_sort_rot(both.astype(jnp.bfloat16))
    srt = srt.astype(jnp.float32)
    for i in range(nb):
        d = srt[:, 2 * i * p:(2 * i + 1) * p] - srt[:, (2 * i + 1) * p:(2 * i + 2) * p]
        s_ref[i] = jnp.sum(d * d, axis=0, keepdims=True)


def kernel(x, y, theta):
    b, n, dd = x.shape
    p = theta.shape[1]
    hi = n >> _LOW
    xs = x.reshape(b, hi, n // hi, dd).transpose(0, 2, 1, 3).reshape(b, n, dd)
    ys = y.reshape(b, hi, n // hi, dd).transpose(0, 2, 1, 3).reshape(b, n, dd)
    theta_t = theta.transpose(0, 2, 1)
    nb = 2
    s = pl.pallas_call(
        _swd_body,
        grid=(b // nb,),
        in_specs=[
            pl.BlockSpec((nb, n, dd), lambda i: (i, 0, 0)),
            pl.BlockSpec((nb, n, dd), lambda i: (i, 0, 0)),
            pl.BlockSpec((nb, dd, p), lambda i: (i, 0, 0)),
        ],
        out_specs=pl.BlockSpec((nb, 1, p), lambda i: (i, 0, 0)),
        out_shape=jax.ShapeDtypeStruct((b, 1, p), jnp.float32),
    )(xs, ys, theta_t)
    distances = jnp.sqrt(jnp.mean(s[:, 0, :], axis=1))
    return jnp.mean(distances)


# speedup vs baseline: 1.1920x; 1.1920x over previous
import jax
import jax.numpy as jnp
from jax.experimental import pallas as pl
from jax.experimental.pallas import tpu as pltpu

_LOW = 7
_TILE = 16


def _cmpex_reshape(w, jp):
    n, c = w.shape
    g = n // (2 * jp)
    w4 = w.reshape(g, 2, jp, c)
    a, b = w4[:, 0], w4[:, 1]
    lo, hi = jnp.minimum(a, b), jnp.maximum(a, b)
    return jnp.concatenate([lo[:, None], hi[:, None]], axis=1).reshape(n, c)


def _cmpex_roll(w, jp, row):
    n, c = w.shape
    w3 = w.reshape(n // _TILE, _TILE, c)
    down = pltpu.roll(w3, _TILE - jp, 1)
    lo = jnp.minimum(w3, down)
    hi = jnp.maximum(w3, down)
    hi_up = pltpu.roll(hi, jp, 1)
    t = jax.lax.broadcasted_iota(jnp.int32, (1, _TILE, 1), 1)
    is_a = (t & jp) == 0
    return jnp.where(is_a, lo, hi_up).reshape(n, c)


def _bitonic_sort_rot(v):
    n = v.shape[0]
    row = jax.lax.broadcasted_iota(jnp.int32, (n, 1), 0)
    i_row = ((row & (_TILE - 1)) << _LOW) | (row >> 4)

    def sign(k):
        if k >= n:
            return None
        return jnp.where((i_row & k) == 0, 1.0, -1.0).astype(v.dtype)

    def cmpex(w, j):
        b = j.bit_length() - 1
        if b < _LOW:
            return _cmpex_reshape(w, _TILE << b)
        return _cmpex_roll(w, 1 << (b - _LOW), row)

    w = v * sign(2)
    k = 2
    while k <= n:
        j = k // 2
        while j >= 1:
            w = cmpex(w, j)
            j //= 2
        s_old, s_new = sign(k), sign(2 * k)
        if s_new is not None:
            w = w * (s_old * s_new)
        elif k < n:
            w = w * s_old
        k *= 2
    return w


def _swd_body(x_ref, y_ref, tht_ref, s_ref):
    nb = x_ref.shape[0]
    p = tht_ref.shape[2]
    projs = []
    for i in range(nb):
        projs.append(jnp.dot(x_ref[i], tht_ref[i],
                             preferred_element_type=jnp.float32))
        projs.append(jnp.dot(y_ref[i], tht_ref[i],
                             preferred_element_type=jnp.float32))
    both = jnp.concatenate(projs, axis=1)
    srt = _bitonic_sort_rot(both.astype(jnp.bfloat16))
    srt = srt.astype(jnp.float32)
    for i in range(nb):
        d = srt[:, 2 * i * p:(2 * i + 1) * p] - srt[:, (2 * i + 1) * p:(2 * i + 2) * p]
        s_ref[i] = jnp.sum(d * d, axis=0, keepdims=True)


def kernel(x, y, theta):
    b, n, dd = x.shape
    p = theta.shape[1]
    hi = n >> _LOW
    xs = x.reshape(b, hi, n // hi, dd).transpose(0, 2, 1, 3).reshape(b, n, dd)
    ys = y.reshape(b, hi, n // hi, dd).transpose(0, 2, 1, 3).reshape(b, n, dd)
    theta_t = theta.transpose(0, 2, 1)
    nb = 1
    s = pl.pallas_call(
        _swd_body,
        grid=(b // nb,),
        in_specs=[
            pl.BlockSpec((nb, n, dd), lambda i: (i, 0, 0)),
            pl.BlockSpec((nb, n, dd), lambda i: (i, 0, 0)),
            pl.BlockSpec((nb, dd, p), lambda i: (i, 0, 0)),
        ],
        out_specs=pl.BlockSpec((nb, 1, p), lambda i: (i, 0, 0)),
        out_shape=jax.ShapeDtypeStruct((b, 1, p), jnp.float32),
    )(xs, ys, theta_t)
    distances = jnp.sqrt(jnp.mean(s[:, 0, :], axis=1))
    return jnp.mean(distances)


# speedup vs baseline: 1.2384x; 1.0390x over previous
import jax
import jax.numpy as jnp
from jax.experimental import pallas as pl
from jax.experimental.pallas import tpu as pltpu

_LOW = 7
_TILE = 16


def _cmpex_reshape(w, jp):
    n, c = w.shape
    g = n // (2 * jp)
    w4 = w.reshape(g, 2, jp, c)
    a, b = w4[:, 0], w4[:, 1]
    lo, hi = jnp.minimum(a, b), jnp.maximum(a, b)
    return jnp.concatenate([lo[:, None], hi[:, None]], axis=1).reshape(n, c)


def _cmpex_roll(w, jp, row):
    n, c = w.shape
    w3 = w.reshape(n // _TILE, _TILE, c)
    down = pltpu.roll(w3, _TILE - jp, 1)
    up = pltpu.roll(w3, jp, 1)
    t = jax.lax.broadcasted_iota(jnp.int32, (1, _TILE, 1), 1)
    is_a = (t & jp) == 0
    lo = jnp.minimum(w3, down)
    hi = jnp.maximum(w3, up)
    return jnp.where(is_a, lo, hi).reshape(n, c)


def _bitonic_sort_rot(v):
    n = v.shape[0]
    row = jax.lax.broadcasted_iota(jnp.int32, (n, 1), 0)
    i_row = ((row & (_TILE - 1)) << _LOW) | (row >> 4)

    def sign(k):
        if k >= n:
            return None
        return jnp.where((i_row & k) == 0, 1.0, -1.0).astype(v.dtype)

    def cmpex(w, j):
        b = j.bit_length() - 1
        if b < _LOW:
            return _cmpex_reshape(w, _TILE << b)
        return _cmpex_roll(w, 1 << (b - _LOW), row)

    w = v * sign(2)
    k = 2
    while k <= n:
        j = k // 2
        while j >= 1:
            w = cmpex(w, j)
            j //= 2
        s_old, s_new = sign(k), sign(2 * k)
        if s_new is not None:
            w = w * (s_old * s_new)
        elif k < n:
            w = w * s_old
        k *= 2
    return w


def _swd_body(x_ref, y_ref, tht_ref, s_ref):
    nb = x_ref.shape[0]
    p = tht_ref.shape[2]
    projs = []
    for i in range(nb):
        projs.append(jnp.dot(x_ref[i], tht_ref[i],
                             preferred_element_type=jnp.float32))
        projs.append(jnp.dot(y_ref[i], tht_ref[i],
                             preferred_element_type=jnp.float32))
    both = jnp.concatenate(projs, axis=1)
    srt = _bitonic_sort_rot(both.astype(jnp.bfloat16))
    srt = srt.astype(jnp.float32)
    for i in range(nb):
        d = srt[:, 2 * i * p:(2 * i + 1) * p] - srt[:, (2 * i + 1) * p:(2 * i + 2) * p]
        s_ref[i] = jnp.sum(d * d, axis=0, keepdims=True)


def kernel(x, y, theta):
    b, n, dd = x.shape
    p = theta.shape[1]
    hi = n >> _LOW
    xs = x.reshape(b, hi, n // hi, dd).transpose(0, 2, 1, 3).reshape(b, n, dd)
    ys = y.reshape(b, hi, n // hi, dd).transpose(0, 2, 1, 3).reshape(b, n, dd)
    theta_t = theta.transpose(0, 2, 1)
    nb = 1
    s = pl.pallas_call(
        _swd_body,
        grid=(b // nb,),
        in_specs=[
            pl.BlockSpec((nb, n, dd), lambda i: (i, 0, 0)),
            pl.BlockSpec((nb, n, dd), lambda i: (i, 0, 0)),
            pl.BlockSpec((nb, dd, p), lambda i: (i, 0, 0)),
        ],
        out_specs=pl.BlockSpec((nb, 1, p), lambda i: (i, 0, 0)),
        out_shape=jax.ShapeDtypeStruct((b, 1, p), jnp.float32),
    )(xs, ys, theta_t)
    distances = jnp.sqrt(jnp.mean(s[:, 0, :], axis=1))
    return jnp.mean(distances)
